# Initial kernel scaffold; baseline (speedup 1.0000x reference)
import jax
import jax.numpy as jnp
from jax.experimental import pallas as pl


def kernel(x, edge_index, batch, num_graphs, Wl1, bl1, Wr1, Wl2, bl2, Wr2, Wl3, bl3, Wr3, Wl4, bl4, Wr4, fc1_W, fc1_b, fc2_W, fc2_b):
    raise NotImplementedError("write your pallas kernel here")


# speedup vs baseline: 3.2752x; 3.2752x over previous
import functools

import jax
import jax.numpy as jnp
from jax import lax
from jax.experimental import pallas as pl
from jax.experimental.pallas import tpu as pltpu
from jax.experimental.pallas import tpu_sc as plsc

N = 10000
F = 128
G = 64
C = 10

NC = 2
NS = 16
NW = NC * NS

N_PAD = 10240
ROWS_PER_TILE = N_PAD // NS
ABSORB = N

K = 128

_MESH = plsc.VectorSubcoreMesh(
    core_axis_name="c", subcore_axis_name="s", num_cores=NC, num_subcores=NS)


def _sc_body_seg(with_cnt, n_chunks, x_hbm, src_hbm, dst_hbm, z128_hbm, z16_hbm,
                 ones_hbm, seg_hbm, cnt_hbm, srcv, dstv, rowsv, accum, sem,
                 onesv, c16v, cntacc):
    cid = lax.axis_index("c")
    sid = lax.axis_index("s")
    row0 = sid * ROWS_PER_TILE

    pltpu.sync_copy(z128_hbm, rowsv)
    for z in range(ROWS_PER_TILE // K):
        pltpu.sync_copy(rowsv, accum.at[pl.ds(row0 + z * K, K)])
    if with_cnt:
        pltpu.sync_copy(z16_hbm, c16v)
        pltpu.sync_copy(c16v, cntacc.at[pl.ds(row0, ROWS_PER_TILE)])
        pltpu.sync_copy(ones_hbm, onesv)
    plsc.subcore_barrier()

    base = (cid * NS + sid) * (n_chunks * K)

    def chunk(j, carry):
        off = base + j * K
        pltpu.sync_copy(src_hbm.at[pl.ds(off, K)], srcv)
        pltpu.sync_copy(dst_hbm.at[pl.ds(off, K)], dstv.at[0])
        pltpu.async_copy(x_hbm.at[srcv], rowsv, sem).wait()
        pltpu.sync_copy(rowsv, accum.at[dstv.at[0]], add=True)
        if with_cnt:
            pltpu.sync_copy(onesv, cntacc.at[dstv.at[0]], add=True)
        return carry

    lax.fori_loop(0, n_chunks, chunk, 0)
    plsc.subcore_barrier()

    for z in range(ROWS_PER_TILE // K):
        sl = pl.ds(row0 + z * K, K)
        pltpu.sync_copy(accum.at[sl], rowsv)
        pltpu.sync_copy(rowsv, seg_hbm.at[cid].at[sl])
    if with_cnt:
        pltpu.sync_copy(cntacc.at[pl.ds(row0, ROWS_PER_TILE)], c16v)
        pltpu.sync_copy(c16v, cnt_hbm.at[cid].at[pl.ds(row0, ROWS_PER_TILE)])


def _make_sc_aggregate(with_cnt, n_chunks):
    seg_t = jax.ShapeDtypeStruct((NC, N_PAD, F), jnp.float32)
    out_type = (seg_t, jax.ShapeDtypeStruct((NC, N_PAD, 16), jnp.float32)) if with_cnt else seg_t
    scratch = [
        pltpu.VMEM((K,), jnp.int32),
        pltpu.VMEM((1, K), jnp.int32),
        pltpu.VMEM((K, F), jnp.float32),
        pltpu.VMEM_SHARED((N_PAD, F), jnp.float32),
        pltpu.SemaphoreType.DMA,
        pltpu.VMEM((K, 16), jnp.float32),
        pltpu.VMEM((ROWS_PER_TILE, 16), jnp.float32),
        pltpu.VMEM_SHARED((N_PAD, 16), jnp.float32),
    ]

    if with_cnt:
        def body(x, src, dst, z128, z16, ones, seg, cnt,
                 srcv, dstv, rowsv, accum, sem, onesv, c16v, cntacc):
            _sc_body_seg(True, n_chunks, x, src, dst, z128, z16, ones, seg, cnt,
                         srcv, dstv, rowsv, accum, sem, onesv, c16v, cntacc)
    else:
        scratch = scratch[:5]

        def body(x, src, dst, z128, seg,
                 srcv, dstv, rowsv, accum, sem):
            _sc_body_seg(False, n_chunks, x, src, dst, z128, None, None, seg,
                         None, srcv, dstv, rowsv, accum, sem, None, None, None)

    return pl.kernel(body, out_type=out_type, mesh=_MESH,
                     scratch_types=tuple(scratch),
                     compiler_params=pltpu.CompilerParams(
                         use_tc_tiling_on_sc=False))


ROW_BLK = 256
N_BLKS = N_PAD // ROW_BLK


def _tc_layer_body(seg_ref, cnt_ref, x_ref, wl_ref, bl_ref, wr_ref, o_ref):
    seg = seg_ref[0] + seg_ref[1]
    cnt = jnp.sum(cnt_ref[0] + cnt_ref[1], axis=1, keepdims=True) * (1.0 / 16.0)
    mean = seg / jnp.maximum(cnt, 1.0)
    dn = (((1,), (1,)), ((), ()))
    h = (lax.dot_general(mean, wl_ref[...], dn, preferred_element_type=jnp.float32)
         + bl_ref[0]
         + lax.dot_general(x_ref[...], wr_ref[...], dn,
                           preferred_element_type=jnp.float32))
    o_ref[...] = jnp.maximum(h, 0.0)


_tc_layer = pl.pallas_call(
    _tc_layer_body,
    grid=(N_BLKS,),
    in_specs=[
        pl.BlockSpec((NC, ROW_BLK, F), lambda i: (0, i, 0)),
        pl.BlockSpec((NC, ROW_BLK, 16), lambda i: (0, i, 0)),
        pl.BlockSpec((ROW_BLK, F), lambda i: (i, 0)),
        pl.BlockSpec((F, F), lambda i: (0, 0)),
        pl.BlockSpec((1, F), lambda i: (0, 0)),
        pl.BlockSpec((F, F), lambda i: (0, 0)),
    ],
    out_specs=pl.BlockSpec((ROW_BLK, F), lambda i: (i, 0)),
    out_shape=jax.ShapeDtypeStruct((N_PAD, F), jnp.float32),
)


def _tc_pool_body(h_ref, b_ref, fc1w_ref, fc1b_ref, fc2w_ref, fc2b_ref, o_ref,
                  sum_scr, cnt_scr):
    i = pl.program_id(0)
    bids = b_ref[0, 0, :]
    onehot = (lax.broadcasted_iota(jnp.int32, (G, ROW_BLK), 0)
              == bids[None, :]).astype(jnp.float32)
    psum = jnp.dot(onehot, h_ref[...], preferred_element_type=jnp.float32)
    pcnt = jnp.broadcast_to(jnp.sum(onehot, axis=1, keepdims=True), (G, F))

    @pl.when(i == 0)
    def _():
        sum_scr[...] = jnp.zeros_like(sum_scr)
        cnt_scr[...] = jnp.zeros_like(cnt_scr)

    sum_scr[...] += psum
    cnt_scr[...] += pcnt

    @pl.when(i == N_BLKS - 1)
    def _():
        pooled = sum_scr[...] / jnp.maximum(cnt_scr[...], 1.0)
        dn = (((1,), (1,)), ((), ()))
        emb = lax.dot_general(pooled, fc1w_ref[...], dn,
                              preferred_element_type=jnp.float32) + fc1b_ref[0]
        o_ref[...] = lax.dot_general(emb, fc2w_ref[...], dn,
                                     preferred_element_type=jnp.float32) + fc2b_ref[0]


_tc_pool = pl.pallas_call(
    _tc_pool_body,
    grid=(N_BLKS,),
    in_specs=[
        pl.BlockSpec((ROW_BLK, F), lambda i: (i, 0)),
        pl.BlockSpec((1, 1, ROW_BLK), lambda i: (i, 0, 0)),
        pl.BlockSpec((F, F), lambda i: (0, 0)),
        pl.BlockSpec((1, F), lambda i: (0, 0)),
        pl.BlockSpec((F, F), lambda i: (0, 0)),
        pl.BlockSpec((1, F), lambda i: (0, 0)),
    ],
    out_specs=pl.BlockSpec((G, F), lambda i: (0, 0)),
    out_shape=jax.ShapeDtypeStruct((G, F), jnp.float32),
    scratch_shapes=[pltpu.VMEM((G, F), jnp.float32),
                    pltpu.VMEM((G, F), jnp.float32)],
)


def kernel(x, edge_index, batch, num_graphs, Wl1, bl1, Wr1, Wl2, bl2, Wr2,
           Wl3, bl3, Wr3, Wl4, bl4, Wr4, fc1_W, fc1_b, fc2_W, fc2_b):
    E = edge_index.shape[1]
    per_tile = -(-E // (NW * K)) * K
    n_chunks = per_tile // K
    e_pad = per_tile * NW

    src = jnp.pad(edge_index[0], (0, e_pad - E))
    dst = jnp.pad(edge_index[1], (0, e_pad - E), constant_values=ABSORB)

    x_pad = jnp.pad(x, ((0, N_PAD - N), (0, 0)))
    batch3 = jnp.pad(batch, (0, N_PAD - N), constant_values=G).reshape(
        N_BLKS, 1, ROW_BLK)

    z128 = jnp.zeros((K, F), jnp.float32)
    z16 = jnp.zeros((ROWS_PER_TILE, 16), jnp.float32)
    ones16 = jnp.ones((K, 16), jnp.float32)

    fc2_Wp = jnp.zeros((F, F), jnp.float32).at[:C].set(fc2_W)
    fc2_bp = jnp.zeros((F,), jnp.float32).at[:C].set(fc2_b)

    sc1 = _make_sc_aggregate(True, n_chunks)

    seg, cnt = sc1(x_pad, src, dst, z128, z16, ones16)
    h = _tc_layer(seg, cnt, x_pad, Wl1, bl1.reshape(1, F), Wr1)
    for wl, bl, wr in ((Wl2, bl2, Wr2), (Wl3, bl3, Wr3), (Wl4, bl4, Wr4)):
        seg, _ = sc1(h, src, dst, z128, z16, ones16)
        h = _tc_layer(seg, cnt, h, wl, bl.reshape(1, F), wr)

    out = _tc_pool(h, batch3, fc1_W, fc1_b.reshape(1, F), fc2_Wp,
                   fc2_bp.reshape(1, F))
    return out[:, :C]
